# baseline (speedup 1.0000x reference)
import functools

import jax
import jax.numpy as jnp
from jax import lax
from jax.experimental import pallas as pl
from jax.experimental.pallas import tpu as pltpu
from jax.experimental.pallas import tpu_sc as plsc

N = 10000
E = 320000
D = 128
G = 64

_NC = 2
_NS = 16
_NW = _NC * _NS
_EPW = E // _NW
_CHUNK = 40
_NCHUNK = _EPW // _CHUNK
_NPAD = 10240
_RPT = _NPAD // _NS
_DEPTH = 4


def _sc_agg_body(r_hbm, e_hbm, out_hbm,
                 e0, e1, e2, e3, r0, r1, r2, r3, agg_sh,
                 is0, is1, is2, is3, gs0, gs1, gs2, gs3,
                 ss0, ss1, ss2, ss3):
    cid = lax.axis_index("c")
    sid = lax.axis_index("s")
    wid = sid * _NC + cid

    ebuf = [e0, e1, e2, e3]
    rows = [r0, r1, r2, r3]
    isem = [is0, is1, is2, is3]
    gsem = [gs0, gs1, gs2, gs3]
    ssem = [ss0, ss1, ss2, ss3]

    zero16 = jnp.zeros((16,), jnp.float32)

    def zrow(i, carry):
        for j in range(D // 16):
            r0[i, pl.ds(j * 16, 16)] = zero16
        return carry

    lax.fori_loop(0, _CHUNK, zrow, 0)

    row0 = sid * _RPT
    for t in range(_RPT // _CHUNK):
        pltpu.sync_copy(r0, agg_sh.at[pl.ds(row0 + t * _CHUNK, _CHUNK)])
    plsc.subcore_barrier()

    def idx_load(g, b):
        pltpu.async_copy(e_hbm.at[wid, g], ebuf[b], isem[b])

    def idx_wait(g, b):
        pltpu.make_async_copy(e_hbm.at[wid, g], ebuf[b], isem[b]).wait()

    def gather_start(b):
        pltpu.async_copy(r_hbm.at[ebuf[b].at[0]], rows[b], gsem[b])

    def gather_wait(b):
        pltpu.make_async_copy(r_hbm.at[ebuf[b].at[0]], rows[b], gsem[b]).wait()

    def scatter_start(b):
        pltpu.async_copy(rows[b], agg_sh.at[ebuf[b].at[1]], ssem[b], add=True)

    def scatter_wait(b):
        pltpu.make_async_copy(rows[b], agg_sh.at[ebuf[b].at[1]], ssem[b]).wait()

    idx_load(0, 0)
    idx_load(1, 1)
    idx_wait(0, 0)
    gather_start(0)
    idx_wait(1, 1)
    gather_start(1)

    def body(k, carry):
        for p in range(_DEPTH):
            g = _DEPTH * k + p
            bn = (p + 2) % _DEPTH

            @pl.when(g >= 2)
            def _():
                scatter_wait(bn)

            @pl.when(g + 2 < _NCHUNK)
            def _():
                idx_load(g + 2, bn)

            @pl.when(g < _NCHUNK)
            def _():
                gather_wait(p)
                scatter_start(p)

            @pl.when(g + 2 < _NCHUNK)
            def _():
                idx_wait(g + 2, bn)
                gather_start(bn)
        return carry

    lax.fori_loop(0, (_NCHUNK + 2 + _DEPTH - 1) // _DEPTH, body, 0)

    plsc.subcore_barrier()
    pltpu.sync_copy(agg_sh.at[pl.ds(row0, _RPT)],
                    out_hbm.at[cid, pl.ds(row0, _RPT)])


_sc_agg = functools.partial(
    pl.kernel,
    mesh=plsc.VectorSubcoreMesh(core_axis_name="c", subcore_axis_name="s"),
    out_type=jax.ShapeDtypeStruct((_NC, _NPAD, D), jnp.float32),
    scratch_types=(
        [pltpu.VMEM((2, _CHUNK), jnp.int32) for _ in range(_DEPTH)]
        + [pltpu.VMEM((_CHUNK, D), jnp.float32) for _ in range(_DEPTH)]
        + [pltpu.VMEM_SHARED((_NPAD, D), jnp.float32)]
        + [pltpu.SemaphoreType.DMA for _ in range(3 * _DEPTH)]
    ),
)(_sc_agg_body)


def _in_body(x_ref, w_ref, b_ref, h_ref, r_ref):
    h = jnp.dot(x_ref[...], w_ref[...],
                preferred_element_type=jnp.float32) + b_ref[...]
    h_ref[...] = h
    r_ref[...] = jnp.maximum(h, 0.0)


_in_call = pl.pallas_call(
    _in_body,
    out_shape=[jax.ShapeDtypeStruct((_NPAD, D), jnp.float32),
               jax.ShapeDtypeStruct((_NPAD, D), jnp.float32)],
)


_MLP_BLK = 1280


def _mlp_body(s_ref, h_ref, a_ref, w1_ref, b1_ref, g_ref, be_ref,
              w2_ref, b2_ref, ho_ref, ro_ref):
    h = h_ref[...]
    z = s_ref[0] * h + a_ref[0] + a_ref[1]
    t = jnp.dot(z, w1_ref[...], preferred_element_type=jnp.float32) + b1_ref[...]
    mu = jnp.mean(t, axis=-1, keepdims=True)
    c = t - mu
    var = jnp.mean(c * c, axis=-1, keepdims=True)
    t = c * lax.rsqrt(var + 1e-5) * g_ref[...] + be_ref[...]
    t = jnp.maximum(t, 0.0)
    u = jnp.dot(t, w2_ref[...], preferred_element_type=jnp.float32) + b2_ref[...]
    hn = h + u
    ho_ref[...] = hn
    ro_ref[...] = jnp.maximum(hn, 0.0)


_mlp_call = pl.pallas_call(
    _mlp_body,
    grid=(_NPAD // _MLP_BLK,),
    in_specs=[
        pl.BlockSpec(memory_space=pltpu.SMEM),
        pl.BlockSpec((_MLP_BLK, D), lambda i: (i, 0)),
        pl.BlockSpec((_NC, _MLP_BLK, D), lambda i: (0, i, 0)),
        pl.BlockSpec((D, 2 * D), lambda i: (0, 0)),
        pl.BlockSpec((1, 2 * D), lambda i: (0, 0)),
        pl.BlockSpec((1, 2 * D), lambda i: (0, 0)),
        pl.BlockSpec((1, 2 * D), lambda i: (0, 0)),
        pl.BlockSpec((2 * D, D), lambda i: (0, 0)),
        pl.BlockSpec((1, D), lambda i: (0, 0)),
    ],
    out_specs=[
        pl.BlockSpec((_MLP_BLK, D), lambda i: (i, 0)),
        pl.BlockSpec((_MLP_BLK, D), lambda i: (i, 0)),
    ],
    out_shape=[jax.ShapeDtypeStruct((_NPAD, D), jnp.float32),
               jax.ShapeDtypeStruct((_NPAD, D), jnp.float32)],
)


def _head_body(b_ref, h_ref, wo1_ref, bo1_ref, wo2_ref, bo2_ref, o_ref):
    seg = b_ref[...]
    gid = lax.broadcasted_iota(jnp.int32, (G, _NPAD), 0)
    onehot = jnp.where(gid == seg, 1.0, 0.0)
    pooled = jnp.dot(onehot, h_ref[...], preferred_element_type=jnp.float32)
    t = jnp.dot(pooled, wo1_ref[...], preferred_element_type=jnp.float32)
    t = jnp.maximum(t + bo1_ref[...], 0.0)
    o_ref[...] = jnp.dot(t, wo2_ref[...],
                         preferred_element_type=jnp.float32) + bo2_ref[...]


_head_call = pl.pallas_call(
    _head_body,
    out_shape=jax.ShapeDtypeStruct((G, D), jnp.float32),
)


def kernel(x, edge_index, batch, W_in, b_in, eps, W1, b1, gamma, beta,
           W2, b2, Wo1, bo1, Wo2, bo2):
    ei = edge_index.astype(jnp.int32).reshape(2, _NW, _NCHUNK, _CHUNK)
    edges = jnp.stack([ei[0], ei[1]], axis=2)
    x_pad = jnp.zeros((_NPAD, D), jnp.float32).at[:N].set(x)
    batch_pad = jnp.concatenate(
        [batch.astype(jnp.int32), jnp.full((_NPAD - N,), G, jnp.int32)]
    ).reshape(1, _NPAD)

    h, r = _in_call(x_pad, W_in, b_in.reshape(1, D))
    for i in range(3):
        agg = _sc_agg(r, edges)
        scale = (1.0 + eps[i]).reshape(1)
        h, r = _mlp_call(scale, h, agg, W1[i], b1[i].reshape(1, 2 * D),
                         gamma[i].reshape(1, 2 * D), beta[i].reshape(1, 2 * D),
                         W2[i], b2[i].reshape(1, D))
    out = _head_call(batch_pad, h, Wo1, bo1.reshape(1, 2 * D),
                     Wo2, bo2.reshape(1, D))
    return out.reshape(-1)


# speedup vs baseline: 1.3378x; 1.3378x over previous
import functools

import jax
import jax.numpy as jnp
from jax import lax
from jax.experimental import pallas as pl
from jax.experimental.pallas import tpu as pltpu
from jax.experimental.pallas import tpu_sc as plsc

N = 10000
E = 320000
D = 128
G = 64

_NC = 2
_NS = 16
_NW = _NC * _NS
_EPW = E // _NW
_CHUNK = 80
_NCHUNK = _EPW // _CHUNK
_NPAD = 10240
_RPT = _NPAD // _NS
_DEPTH = 3


def _sc_agg_body(r_hbm, e_hbm, out_hbm,
                 e0, e1, e2, r0, r1, r2, agg_sh,
                 is0, is1, is2, gs0, gs1, gs2,
                 ss0, ss1, ss2):
    cid = lax.axis_index("c")
    sid = lax.axis_index("s")
    wid = sid * _NC + cid

    ebuf = [e0, e1, e2]
    rows = [r0, r1, r2]
    isem = [is0, is1, is2]
    gsem = [gs0, gs1, gs2]
    ssem = [ss0, ss1, ss2]

    zero16 = jnp.zeros((16,), jnp.float32)

    def zrow(i, carry):
        for j in range(D // 16):
            r0[i, pl.ds(j * 16, 16)] = zero16
        return carry

    lax.fori_loop(0, _CHUNK, zrow, 0)

    row0 = sid * _RPT
    for t in range(_RPT // _CHUNK):
        pltpu.sync_copy(r0, agg_sh.at[pl.ds(row0 + t * _CHUNK, _CHUNK)])
    plsc.subcore_barrier()

    def idx_load(g, b):
        pltpu.async_copy(e_hbm.at[wid, g], ebuf[b], isem[b])

    def idx_wait(g, b):
        pltpu.make_async_copy(e_hbm.at[wid, g], ebuf[b], isem[b]).wait()

    def gather_start(b):
        pltpu.async_copy(r_hbm.at[ebuf[b].at[0]], rows[b], gsem[b])

    def gather_wait(b):
        pltpu.make_async_copy(r_hbm.at[ebuf[b].at[0]], rows[b], gsem[b]).wait()

    def scatter_start(b):
        pltpu.async_copy(rows[b], agg_sh.at[ebuf[b].at[1]], ssem[b], add=True)

    def scatter_wait(b):
        pltpu.make_async_copy(rows[b], agg_sh.at[ebuf[b].at[1]], ssem[b]).wait()

    def phase(g, p, bn, first, tail):
        if not first:
            scatter_wait(bn)
        if tail < 1:
            idx_load(g + 2, bn)
        if tail < 3:
            gather_wait(p)
            scatter_start(p)
        if tail < 1:
            idx_wait(g + 2, bn)
            gather_start(bn)

    idx_load(0, 0)
    idx_load(1, 1)
    idx_wait(0, 0)
    gather_start(0)
    idx_wait(1, 1)
    gather_start(1)

    phase(0, 0, 2, True, 0)
    phase(1, 1, 0, False, 0)

    def body(k, carry):
        g = 3 * k + 2
        phase(g, 2, 1, False, 0)
        phase(g + 1, 0, 2, False, 0)
        phase(g + 2, 1, 0, False, 0)
        return carry

    lax.fori_loop(0, (_NCHUNK - 5) // 3, body, 0)

    phase(_NCHUNK - 3, (_NCHUNK - 3) % 3, (_NCHUNK - 1) % 3, False, 0)
    phase(_NCHUNK - 2, (_NCHUNK - 2) % 3, _NCHUNK % 3, False, 1)
    phase(_NCHUNK - 1, (_NCHUNK - 1) % 3, (_NCHUNK + 1) % 3, False, 2)
    phase(_NCHUNK, _NCHUNK % 3, (_NCHUNK + 2) % 3, False, 3)

    plsc.subcore_barrier()
    pltpu.sync_copy(agg_sh.at[pl.ds(row0, _RPT)],
                    out_hbm.at[cid, pl.ds(row0, _RPT)])


_sc_agg = functools.partial(
    pl.kernel,
    mesh=plsc.VectorSubcoreMesh(core_axis_name="c", subcore_axis_name="s"),
    out_type=jax.ShapeDtypeStruct((_NC, _NPAD, D), jnp.float32),
    scratch_types=(
        [pltpu.VMEM((2, _CHUNK), jnp.int32) for _ in range(_DEPTH)]
        + [pltpu.VMEM((_CHUNK, D), jnp.float32) for _ in range(_DEPTH)]
        + [pltpu.VMEM_SHARED((_NPAD, D), jnp.float32)]
        + [pltpu.SemaphoreType.DMA for _ in range(3 * _DEPTH)]
    ),
)(_sc_agg_body)


def _in_body(x_ref, w_ref, b_ref, h_ref, r_ref):
    h = jnp.dot(x_ref[...], w_ref[...],
                preferred_element_type=jnp.float32) + b_ref[...]
    h_ref[...] = h
    r_ref[...] = jnp.maximum(h, 0.0)


_in_call = pl.pallas_call(
    _in_body,
    out_shape=[jax.ShapeDtypeStruct((_NPAD, D), jnp.float32),
               jax.ShapeDtypeStruct((_NPAD, D), jnp.float32)],
)


_MLP_BLK = 1280


def _mlp_body(s_ref, h_ref, a_ref, w1_ref, b1_ref, g_ref, be_ref,
              w2_ref, b2_ref, ho_ref, ro_ref):
    h = h_ref[...]
    z = s_ref[0] * h + a_ref[0] + a_ref[1]
    t = jnp.dot(z, w1_ref[...], preferred_element_type=jnp.float32) + b1_ref[...]
    mu = jnp.mean(t, axis=-1, keepdims=True)
    c = t - mu
    var = jnp.mean(c * c, axis=-1, keepdims=True)
    t = c * lax.rsqrt(var + 1e-5) * g_ref[...] + be_ref[...]
    t = jnp.maximum(t, 0.0)
    u = jnp.dot(t, w2_ref[...], preferred_element_type=jnp.float32) + b2_ref[...]
    hn = h + u
    ho_ref[...] = hn
    ro_ref[...] = jnp.maximum(hn, 0.0)


_mlp_call = pl.pallas_call(
    _mlp_body,
    grid=(_NPAD // _MLP_BLK,),
    in_specs=[
        pl.BlockSpec(memory_space=pltpu.SMEM),
        pl.BlockSpec((_MLP_BLK, D), lambda i: (i, 0)),
        pl.BlockSpec((_NC, _MLP_BLK, D), lambda i: (0, i, 0)),
        pl.BlockSpec((D, 2 * D), lambda i: (0, 0)),
        pl.BlockSpec((1, 2 * D), lambda i: (0, 0)),
        pl.BlockSpec((1, 2 * D), lambda i: (0, 0)),
        pl.BlockSpec((1, 2 * D), lambda i: (0, 0)),
        pl.BlockSpec((2 * D, D), lambda i: (0, 0)),
        pl.BlockSpec((1, D), lambda i: (0, 0)),
    ],
    out_specs=[
        pl.BlockSpec((_MLP_BLK, D), lambda i: (i, 0)),
        pl.BlockSpec((_MLP_BLK, D), lambda i: (i, 0)),
    ],
    out_shape=[jax.ShapeDtypeStruct((_NPAD, D), jnp.float32),
               jax.ShapeDtypeStruct((_NPAD, D), jnp.float32)],
)


def _head_body(b_ref, h_ref, wo1_ref, bo1_ref, wo2_ref, bo2_ref, o_ref):
    seg = b_ref[...]
    gid = lax.broadcasted_iota(jnp.int32, (G, _NPAD), 0)
    onehot = jnp.where(gid == seg, 1.0, 0.0)
    pooled = jnp.dot(onehot, h_ref[...], preferred_element_type=jnp.float32)
    t = jnp.dot(pooled, wo1_ref[...], preferred_element_type=jnp.float32)
    t = jnp.maximum(t + bo1_ref[...], 0.0)
    o_ref[...] = jnp.dot(t, wo2_ref[...],
                         preferred_element_type=jnp.float32) + bo2_ref[...]


_head_call = pl.pallas_call(
    _head_body,
    out_shape=jax.ShapeDtypeStruct((G, D), jnp.float32),
)


def kernel(x, edge_index, batch, W_in, b_in, eps, W1, b1, gamma, beta,
           W2, b2, Wo1, bo1, Wo2, bo2):
    ei = edge_index.astype(jnp.int32).reshape(2, _NW, _NCHUNK, _CHUNK)
    edges = jnp.stack([ei[0], ei[1]], axis=2)
    x_pad = jnp.zeros((_NPAD, D), jnp.float32).at[:N].set(x)
    batch_pad = jnp.concatenate(
        [batch.astype(jnp.int32), jnp.full((_NPAD - N,), G, jnp.int32)]
    ).reshape(1, _NPAD)

    h, r = _in_call(x_pad, W_in, b_in.reshape(1, D))
    for i in range(3):
        agg = _sc_agg(r, edges)
        scale = (1.0 + eps[i]).reshape(1)
        h, r = _mlp_call(scale, h, agg, W1[i], b1[i].reshape(1, 2 * D),
                         gamma[i].reshape(1, 2 * D), beta[i].reshape(1, 2 * D),
                         W2[i], b2[i].reshape(1, D))
    out = _head_call(batch_pad, h, Wo1, bo1.reshape(1, 2 * D),
                     Wo2, bo2.reshape(1, D))
    return out.reshape(-1)


# speedup vs baseline: 1.3886x; 1.0380x over previous
import functools

import jax
import jax.numpy as jnp
from jax import lax
from jax.experimental import pallas as pl
from jax.experimental.pallas import tpu as pltpu
from jax.experimental.pallas import tpu_sc as plsc

N = 10000
E = 320000
D = 128
G = 64

_NC = 2
_NS = 16
_NW = _NC * _NS
_EPW = E // _NW
_CHUNK = 80
_NCHUNK = _EPW // _CHUNK
_NPAD = 10240
_RPT = _NPAD // _NS
_DEPTH = 3


def _sc_agg_body(r_hbm, s_hbm, d_hbm, out_hbm,
                 s0, s1, s2, d0, d1, d2, r0, r1, r2, agg_sh,
                 is0, is1, is2, gs0, gs1, gs2,
                 ss0, ss1, ss2):
    cid = lax.axis_index("c")
    sid = lax.axis_index("s")
    wid = sid * _NC + cid

    sbuf = [s0, s1, s2]
    dbuf = [d0, d1, d2]
    rows = [r0, r1, r2]
    isem = [is0, is1, is2]
    gsem = [gs0, gs1, gs2]
    ssem = [ss0, ss1, ss2]

    zero16 = jnp.zeros((16,), jnp.float32)

    def zrow(i, carry):
        for j in range(D // 16):
            r0[i, pl.ds(j * 16, 16)] = zero16
        return carry

    lax.fori_loop(0, _CHUNK, zrow, 0)

    row0 = sid * _RPT
    for t in range(_RPT // _CHUNK):
        pltpu.sync_copy(r0, agg_sh.at[pl.ds(row0 + t * _CHUNK, _CHUNK)])
    plsc.subcore_barrier()

    def idx_load(g, b):
        pltpu.async_copy(s_hbm.at[wid, g], sbuf[b], isem[b])
        pltpu.async_copy(d_hbm.at[wid, g], dbuf[b], isem[b])

    def idx_wait(g, b):
        pltpu.make_async_copy(s_hbm.at[wid, g], sbuf[b], isem[b]).wait()
        pltpu.make_async_copy(d_hbm.at[wid, g], dbuf[b], isem[b]).wait()

    def gather_start(b):
        pltpu.async_copy(r_hbm.at[sbuf[b].at[0]], rows[b], gsem[b])

    def gather_wait(b):
        pltpu.make_async_copy(r_hbm.at[sbuf[b].at[0]], rows[b], gsem[b]).wait()

    def scatter_start(b):
        pltpu.async_copy(rows[b], agg_sh.at[dbuf[b].at[0]], ssem[b], add=True)

    def scatter_wait(b):
        pltpu.make_async_copy(rows[b], agg_sh.at[dbuf[b].at[0]], ssem[b]).wait()

    def phase(g, p, bn, first, tail):
        if not first:
            scatter_wait(bn)
        if tail < 1:
            idx_load(g + 2, bn)
        if tail < 3:
            gather_wait(p)
            scatter_start(p)
        if tail < 1:
            idx_wait(g + 2, bn)
            gather_start(bn)

    idx_load(0, 0)
    idx_load(1, 1)
    idx_wait(0, 0)
    gather_start(0)
    idx_wait(1, 1)
    gather_start(1)

    phase(0, 0, 2, True, 0)
    phase(1, 1, 0, False, 0)

    def body(k, carry):
        g = 3 * k + 2
        phase(g, 2, 1, False, 0)
        phase(g + 1, 0, 2, False, 0)
        phase(g + 2, 1, 0, False, 0)
        return carry

    lax.fori_loop(0, (_NCHUNK - 5) // 3, body, 0)

    phase(_NCHUNK - 3, (_NCHUNK - 3) % 3, (_NCHUNK - 1) % 3, False, 0)
    phase(_NCHUNK - 2, (_NCHUNK - 2) % 3, _NCHUNK % 3, False, 1)
    phase(_NCHUNK - 1, (_NCHUNK - 1) % 3, (_NCHUNK + 1) % 3, False, 2)
    phase(_NCHUNK, _NCHUNK % 3, (_NCHUNK + 2) % 3, False, 3)

    plsc.subcore_barrier()
    pltpu.sync_copy(agg_sh.at[pl.ds(row0, _RPT)],
                    out_hbm.at[cid, pl.ds(row0, _RPT)])


_sc_agg = functools.partial(
    pl.kernel,
    mesh=plsc.VectorSubcoreMesh(core_axis_name="c", subcore_axis_name="s"),
    out_type=jax.ShapeDtypeStruct((_NC, _NPAD, D), jnp.float32),
    scratch_types=(
        [pltpu.VMEM((1, _CHUNK), jnp.int32) for _ in range(2 * _DEPTH)]
        + [pltpu.VMEM((_CHUNK, D), jnp.float32) for _ in range(_DEPTH)]
        + [pltpu.VMEM_SHARED((_NPAD, D), jnp.float32)]
        + [pltpu.SemaphoreType.DMA for _ in range(3 * _DEPTH)]
    ),
)(_sc_agg_body)


def _in_body(x_ref, w_ref, b_ref, h_ref, r_ref):
    h = jnp.dot(x_ref[...], w_ref[...],
                preferred_element_type=jnp.float32) + b_ref[...]
    h_ref[...] = h
    r_ref[...] = jnp.maximum(h, 0.0)


_in_call = pl.pallas_call(
    _in_body,
    out_shape=[jax.ShapeDtypeStruct((N, D), jnp.float32),
               jax.ShapeDtypeStruct((N, D), jnp.float32)],
)


_MLP_BLK = 2000


def _mlp_body(s_ref, h_ref, a_ref, w1_ref, b1_ref, g_ref, be_ref,
              w2_ref, b2_ref, ho_ref, ro_ref):
    h = h_ref[...]
    z = s_ref[0] * h + a_ref[0] + a_ref[1]
    t = jnp.dot(z, w1_ref[...], preferred_element_type=jnp.float32) + b1_ref[...]
    mu = jnp.mean(t, axis=-1, keepdims=True)
    c = t - mu
    var = jnp.mean(c * c, axis=-1, keepdims=True)
    t = c * lax.rsqrt(var + 1e-5) * g_ref[...] + be_ref[...]
    t = jnp.maximum(t, 0.0)
    u = jnp.dot(t, w2_ref[...], preferred_element_type=jnp.float32) + b2_ref[...]
    hn = h + u
    ho_ref[...] = hn
    ro_ref[...] = jnp.maximum(hn, 0.0)


_mlp_call = pl.pallas_call(
    _mlp_body,
    grid=(N // _MLP_BLK,),
    in_specs=[
        pl.BlockSpec(memory_space=pltpu.SMEM),
        pl.BlockSpec((_MLP_BLK, D), lambda i: (i, 0)),
        pl.BlockSpec((_NC, _MLP_BLK, D), lambda i: (0, i, 0)),
        pl.BlockSpec((D, 2 * D), lambda i: (0, 0)),
        pl.BlockSpec((1, 2 * D), lambda i: (0, 0)),
        pl.BlockSpec((1, 2 * D), lambda i: (0, 0)),
        pl.BlockSpec((1, 2 * D), lambda i: (0, 0)),
        pl.BlockSpec((2 * D, D), lambda i: (0, 0)),
        pl.BlockSpec((1, D), lambda i: (0, 0)),
    ],
    out_specs=[
        pl.BlockSpec((_MLP_BLK, D), lambda i: (i, 0)),
        pl.BlockSpec((_MLP_BLK, D), lambda i: (i, 0)),
    ],
    out_shape=[jax.ShapeDtypeStruct((N, D), jnp.float32),
               jax.ShapeDtypeStruct((N, D), jnp.float32)],
)


def _head_body(b_ref, h_ref, wo1_ref, bo1_ref, wo2_ref, bo2_ref, o_ref):
    seg = b_ref[...]
    gid = lax.broadcasted_iota(jnp.int32, (G, N), 0)
    onehot = jnp.where(gid == seg, 1.0, 0.0)
    pooled = jnp.dot(onehot, h_ref[...], preferred_element_type=jnp.float32)
    t = jnp.dot(pooled, wo1_ref[...], preferred_element_type=jnp.float32)
    t = jnp.maximum(t + bo1_ref[...], 0.0)
    o_ref[...] = jnp.dot(t, wo2_ref[...],
                         preferred_element_type=jnp.float32) + bo2_ref[...]


_head_call = pl.pallas_call(
    _head_body,
    out_shape=jax.ShapeDtypeStruct((G, D), jnp.float32),
)


def kernel(x, edge_index, batch, W_in, b_in, eps, W1, b1, gamma, beta,
           W2, b2, Wo1, bo1, Wo2, bo2):
    src = edge_index[0].astype(jnp.int32).reshape(_NW, _NCHUNK, 1, _CHUNK)
    dst = edge_index[1].astype(jnp.int32).reshape(_NW, _NCHUNK, 1, _CHUNK)
    batch2d = batch.astype(jnp.int32).reshape(1, N)

    h, r = _in_call(x, W_in, b_in.reshape(1, D))
    for i in range(3):
        agg = _sc_agg(r, src, dst)
        scale = (1.0 + eps[i]).reshape(1)
        h, r = _mlp_call(scale, h, agg, W1[i], b1[i].reshape(1, 2 * D),
                         gamma[i].reshape(1, 2 * D), beta[i].reshape(1, 2 * D),
                         W2[i], b2[i].reshape(1, D))
    out = _head_call(batch2d, h, Wo1, bo1.reshape(1, 2 * D),
                     Wo2, bo2.reshape(1, D))
    return out.reshape(-1)


# speedup vs baseline: 1.4119x; 1.0168x over previous
import functools

import jax
import jax.numpy as jnp
from jax import lax
from jax.experimental import pallas as pl
from jax.experimental.pallas import tpu as pltpu
from jax.experimental.pallas import tpu_sc as plsc

N = 10000
E = 320000
D = 128
G = 64

_NC = 2
_NS = 16
_NW = _NC * _NS
_EPW = E // _NW
_CHUNK = 80
_NCHUNK = _EPW // _CHUNK
_NPAD = 10240
_RPT = _NPAD // _NS
_DEPTH = 3


def _sc_agg_body(r_hbm, s_hbm, d_hbm, out_hbm,
                 s0, s1, s2, d0, d1, d2, r0, r1, r2, agg_sh,
                 is0, is1, is2, gs0, gs1, gs2,
                 ss0, ss1, ss2):
    cid = lax.axis_index("c")
    sid = lax.axis_index("s")
    wid = sid * _NC + cid

    sbuf = [s0, s1, s2]
    dbuf = [d0, d1, d2]
    rows = [r0, r1, r2]
    isem = [is0, is1, is2]
    gsem = [gs0, gs1, gs2]
    ssem = [ss0, ss1, ss2]

    zero16 = jnp.zeros((16,), jnp.float32)

    def zrow(i, carry):
        for j in range(D // 16):
            r0[i, pl.ds(j * 16, 16)] = zero16
        return carry

    lax.fori_loop(0, _CHUNK, zrow, 0)

    row0 = sid * _RPT
    for t in range(_RPT // _CHUNK):
        pltpu.sync_copy(r0, agg_sh.at[pl.ds(row0 + t * _CHUNK, _CHUNK)])
    plsc.subcore_barrier()

    def idx_load(g, b):
        pltpu.async_copy(s_hbm.at[wid, g], sbuf[b], isem[b])
        pltpu.async_copy(d_hbm.at[wid, g], dbuf[b], isem[b])

    def idx_wait(g, b):
        pltpu.make_async_copy(s_hbm.at[wid, g], sbuf[b], isem[b]).wait()
        pltpu.make_async_copy(d_hbm.at[wid, g], dbuf[b], isem[b]).wait()

    def gather_start(b):
        pltpu.async_copy(r_hbm.at[sbuf[b].at[0]], rows[b], gsem[b])

    def gather_wait(b):
        pltpu.make_async_copy(r_hbm.at[sbuf[b].at[0]], rows[b], gsem[b]).wait()

    def scatter_start(b):
        pltpu.async_copy(rows[b], agg_sh.at[dbuf[b].at[0]], ssem[b], add=True)

    def scatter_wait(b):
        pltpu.make_async_copy(rows[b], agg_sh.at[dbuf[b].at[0]], ssem[b]).wait()

    def phase(g, p, bn, first, tail):
        if not first:
            scatter_wait(bn)
        if tail < 1:
            idx_load(g + 2, bn)
        if tail < 3:
            gather_wait(p)
            scatter_start(p)
        if tail < 1:
            idx_wait(g + 2, bn)
            gather_start(bn)

    idx_load(0, 0)
    idx_load(1, 1)
    idx_wait(0, 0)
    gather_start(0)
    idx_wait(1, 1)
    gather_start(1)

    phase(0, 0, 2, True, 0)
    phase(1, 1, 0, False, 0)

    def body(k, carry):
        g = 3 * k + 2
        phase(g, 2, 1, False, 0)
        phase(g + 1, 0, 2, False, 0)
        phase(g + 2, 1, 0, False, 0)
        return carry

    lax.fori_loop(0, (_NCHUNK - 5) // 3, body, 0)

    phase(_NCHUNK - 3, (_NCHUNK - 3) % 3, (_NCHUNK - 1) % 3, False, 0)
    phase(_NCHUNK - 2, (_NCHUNK - 2) % 3, _NCHUNK % 3, False, 1)
    phase(_NCHUNK - 1, (_NCHUNK - 1) % 3, (_NCHUNK + 1) % 3, False, 2)
    phase(_NCHUNK, _NCHUNK % 3, (_NCHUNK + 2) % 3, False, 3)

    plsc.subcore_barrier()
    pltpu.sync_copy(agg_sh.at[pl.ds(row0, _RPT)],
                    out_hbm.at[cid, pl.ds(row0, _RPT)])


_sc_agg = functools.partial(
    pl.kernel,
    mesh=plsc.VectorSubcoreMesh(core_axis_name="c", subcore_axis_name="s"),
    out_type=jax.ShapeDtypeStruct((_NC, _NPAD, D), jnp.float32),
    scratch_types=(
        [pltpu.VMEM((1, _CHUNK), jnp.int32) for _ in range(2 * _DEPTH)]
        + [pltpu.VMEM((_CHUNK, D), jnp.float32) for _ in range(_DEPTH)]
        + [pltpu.VMEM_SHARED((_NPAD, D), jnp.float32)]
        + [pltpu.SemaphoreType.DMA for _ in range(3 * _DEPTH)]
    ),
)(_sc_agg_body)


def _in_body(x_ref, w_ref, b_ref, h_ref, r_ref):
    h = jnp.dot(x_ref[...], w_ref[...],
                preferred_element_type=jnp.float32) + b_ref[...]
    h_ref[...] = h
    r_ref[...] = jnp.maximum(h, 0.0)


_in_call = pl.pallas_call(
    _in_body,
    out_shape=[jax.ShapeDtypeStruct((N, D), jnp.float32),
               jax.ShapeDtypeStruct((N, D), jnp.float32)],
)


_MLP_BLK = 2000


def _mlp_body(s_ref, h_ref, a_ref, w1_ref, b1_ref, g_ref, be_ref,
              w2_ref, b2_ref, ho_ref, ro_ref):
    h = h_ref[...]
    z = s_ref[0] * h + a_ref[0] + a_ref[1]
    t = jnp.dot(z, w1_ref[...], preferred_element_type=jnp.float32) + b1_ref[...]
    mu = jnp.mean(t, axis=-1, keepdims=True)
    c = t - mu
    var = jnp.mean(c * c, axis=-1, keepdims=True)
    t = c * lax.rsqrt(var + 1e-5) * g_ref[...] + be_ref[...]
    t = jnp.maximum(t, 0.0)
    u = jnp.dot(t, w2_ref[...], preferred_element_type=jnp.float32) + b2_ref[...]
    hn = h + u
    ho_ref[...] = hn
    ro_ref[...] = jnp.maximum(hn, 0.0)


_mlp_call = pl.pallas_call(
    _mlp_body,
    grid=(N // _MLP_BLK,),
    in_specs=[
        pl.BlockSpec(memory_space=pltpu.SMEM),
        pl.BlockSpec((_MLP_BLK, D), lambda i: (i, 0)),
        pl.BlockSpec((_NC, _MLP_BLK, D), lambda i: (0, i, 0)),
        pl.BlockSpec((D, 2 * D), lambda i: (0, 0)),
        pl.BlockSpec((1, 2 * D), lambda i: (0, 0)),
        pl.BlockSpec((1, 2 * D), lambda i: (0, 0)),
        pl.BlockSpec((1, 2 * D), lambda i: (0, 0)),
        pl.BlockSpec((2 * D, D), lambda i: (0, 0)),
        pl.BlockSpec((1, D), lambda i: (0, 0)),
    ],
    out_specs=[
        pl.BlockSpec((_MLP_BLK, D), lambda i: (i, 0)),
        pl.BlockSpec((_MLP_BLK, D), lambda i: (i, 0)),
    ],
    out_shape=[jax.ShapeDtypeStruct((N, D), jnp.float32),
               jax.ShapeDtypeStruct((N, D), jnp.float32)],
)


def _mlp3_body(s_ref, h_ref, a_ref, w1_ref, b1_ref, g_ref, be_ref,
               w2_ref, b2_ref, b2d_ref, wo1_ref, bo1_ref, wo2_ref, bo2_ref,
               o_ref, acc_ref):
    i = pl.program_id(0)
    h = h_ref[...]
    z = s_ref[0] * h + a_ref[0] + a_ref[1]
    t = jnp.dot(z, w1_ref[...], preferred_element_type=jnp.float32) + b1_ref[...]
    mu = jnp.mean(t, axis=-1, keepdims=True)
    c = t - mu
    var = jnp.mean(c * c, axis=-1, keepdims=True)
    t = c * lax.rsqrt(var + 1e-5) * g_ref[...] + be_ref[...]
    t = jnp.maximum(t, 0.0)
    u = jnp.dot(t, w2_ref[...], preferred_element_type=jnp.float32) + b2_ref[...]
    hn = h + u
    seg = b2d_ref[0]
    gid = lax.broadcasted_iota(jnp.int32, (G, _MLP_BLK), 0)
    onehot = jnp.where(gid == seg, 1.0, 0.0)
    pooled = jnp.dot(onehot, hn, preferred_element_type=jnp.float32)

    @pl.when(i == 0)
    def _():
        acc_ref[...] = pooled

    @pl.when(i > 0)
    def _():
        acc_ref[...] = acc_ref[...] + pooled

    @pl.when(i == N // _MLP_BLK - 1)
    def _():
        t2 = jnp.dot(acc_ref[...], wo1_ref[...],
                     preferred_element_type=jnp.float32) + bo1_ref[...]
        t2 = jnp.maximum(t2, 0.0)
        o_ref[...] = jnp.dot(t2, wo2_ref[...],
                             preferred_element_type=jnp.float32) + bo2_ref[...]


_mlp3_call = pl.pallas_call(
    _mlp3_body,
    grid=(N // _MLP_BLK,),
    in_specs=[
        pl.BlockSpec(memory_space=pltpu.SMEM),
        pl.BlockSpec((_MLP_BLK, D), lambda i: (i, 0)),
        pl.BlockSpec((_NC, _MLP_BLK, D), lambda i: (0, i, 0)),
        pl.BlockSpec((D, 2 * D), lambda i: (0, 0)),
        pl.BlockSpec((1, 2 * D), lambda i: (0, 0)),
        pl.BlockSpec((1, 2 * D), lambda i: (0, 0)),
        pl.BlockSpec((1, 2 * D), lambda i: (0, 0)),
        pl.BlockSpec((2 * D, D), lambda i: (0, 0)),
        pl.BlockSpec((1, D), lambda i: (0, 0)),
        pl.BlockSpec((1, 1, _MLP_BLK), lambda i: (i, 0, 0)),
        pl.BlockSpec((D, 2 * D), lambda i: (0, 0)),
        pl.BlockSpec((1, 2 * D), lambda i: (0, 0)),
        pl.BlockSpec((2 * D, D), lambda i: (0, 0)),
        pl.BlockSpec((1, D), lambda i: (0, 0)),
    ],
    out_specs=pl.BlockSpec((G, D), lambda i: (0, 0)),
    out_shape=jax.ShapeDtypeStruct((G, D), jnp.float32),
    scratch_shapes=[pltpu.VMEM((G, D), jnp.float32)],
)


def kernel(x, edge_index, batch, W_in, b_in, eps, W1, b1, gamma, beta,
           W2, b2, Wo1, bo1, Wo2, bo2):
    src = edge_index[0].astype(jnp.int32).reshape(_NW, _NCHUNK, 1, _CHUNK)
    dst = edge_index[1].astype(jnp.int32).reshape(_NW, _NCHUNK, 1, _CHUNK)
    batch2d = batch.astype(jnp.int32).reshape(N // _MLP_BLK, 1, _MLP_BLK)

    h, r = _in_call(x, W_in, b_in.reshape(1, D))
    for i in range(2):
        agg = _sc_agg(r, src, dst)
        scale = (1.0 + eps[i]).reshape(1)
        h, r = _mlp_call(scale, h, agg, W1[i], b1[i].reshape(1, 2 * D),
                         gamma[i].reshape(1, 2 * D), beta[i].reshape(1, 2 * D),
                         W2[i], b2[i].reshape(1, D))
    agg = _sc_agg(r, src, dst)
    scale = (1.0 + eps[2]).reshape(1)
    out = _mlp3_call(scale, h, agg, W1[2], b1[2].reshape(1, 2 * D),
                     gamma[2].reshape(1, 2 * D), beta[2].reshape(1, 2 * D),
                     W2[2], b2[2].reshape(1, D), batch2d,
                     Wo1, bo1.reshape(1, 2 * D), Wo2, bo2.reshape(1, D))
    return out.reshape(-1)


# speedup vs baseline: 1.4119x; 1.0000x over previous
import functools

import jax
import jax.numpy as jnp
from jax import lax
from jax.experimental import pallas as pl
from jax.experimental.pallas import tpu as pltpu
from jax.experimental.pallas import tpu_sc as plsc

N = 10000
E = 320000
D = 128
G = 64

_NC = 2
_NS = 16
_NW = _NC * _NS
_EPW = E // _NW
_CHUNK = 80
_NCHUNK = _EPW // _CHUNK
_NPAD = 10240
_RPT = _NPAD // _NS
_DEPTH = 3


def _sc_agg_body(r_hbm, s_hbm, d_hbm, out_hbm,
                 s0, s1, s2, d0, d1, d2, r0, r1, r2, agg_sh,
                 is0, is1, is2, gs0, gs1, gs2,
                 ss0, ss1, ss2):
    cid = lax.axis_index("c")
    sid = lax.axis_index("s")
    wid = sid * _NC + cid

    sbuf = [s0, s1, s2]
    dbuf = [d0, d1, d2]
    rows = [r0, r1, r2]
    isem = [is0, is1, is2]
    gsem = [gs0, gs1, gs2]
    ssem = [ss0, ss1, ss2]

    zero16 = jnp.zeros((16,), jnp.float32)

    def zrow(i, carry):
        for j in range(D // 16):
            r0[i, pl.ds(j * 16, 16)] = zero16
        return carry

    lax.fori_loop(0, _CHUNK, zrow, 0)

    row0 = sid * _RPT
    for t in range(_RPT // _CHUNK):
        pltpu.sync_copy(r0, agg_sh.at[pl.ds(row0 + t * _CHUNK, _CHUNK)])
    plsc.subcore_barrier()

    def idx_load(g, b):
        pltpu.async_copy(s_hbm.at[wid, g], sbuf[b], isem[b])
        pltpu.async_copy(d_hbm.at[wid, g], dbuf[b], isem[b])

    def idx_wait(g, b):
        pltpu.make_async_copy(s_hbm.at[wid, g], sbuf[b], isem[b]).wait()
        pltpu.make_async_copy(d_hbm.at[wid, g], dbuf[b], isem[b]).wait()

    _H = _CHUNK // 2

    def gather_start(b):
        pltpu.async_copy(r_hbm.at[sbuf[b].at[0, pl.ds(0, _H)]],
                         rows[b].at[pl.ds(0, _H)], gsem[b])
        pltpu.async_copy(r_hbm.at[sbuf[b].at[0, pl.ds(_H, _H)]],
                         rows[b].at[pl.ds(_H, _H)], gsem[b])

    def gather_wait(b):
        pltpu.make_async_copy(r_hbm.at[sbuf[b].at[0, pl.ds(0, _H)]],
                              rows[b].at[pl.ds(0, _H)], gsem[b]).wait()
        pltpu.make_async_copy(r_hbm.at[sbuf[b].at[0, pl.ds(_H, _H)]],
                              rows[b].at[pl.ds(_H, _H)], gsem[b]).wait()

    def scatter_start(b):
        pltpu.async_copy(rows[b], agg_sh.at[dbuf[b].at[0]], ssem[b], add=True)

    def scatter_wait(b):
        pltpu.make_async_copy(rows[b], agg_sh.at[dbuf[b].at[0]], ssem[b]).wait()

    def phase(g, p, bn, first, tail):
        if not first:
            scatter_wait(bn)
        if tail < 1:
            idx_load(g + 2, bn)
        if tail < 3:
            gather_wait(p)
            scatter_start(p)
        if tail < 1:
            idx_wait(g + 2, bn)
            gather_start(bn)

    idx_load(0, 0)
    idx_load(1, 1)
    idx_wait(0, 0)
    gather_start(0)
    idx_wait(1, 1)
    gather_start(1)

    phase(0, 0, 2, True, 0)
    phase(1, 1, 0, False, 0)

    def body(k, carry):
        g = 3 * k + 2
        phase(g, 2, 1, False, 0)
        phase(g + 1, 0, 2, False, 0)
        phase(g + 2, 1, 0, False, 0)
        return carry

    lax.fori_loop(0, (_NCHUNK - 5) // 3, body, 0)

    phase(_NCHUNK - 3, (_NCHUNK - 3) % 3, (_NCHUNK - 1) % 3, False, 0)
    phase(_NCHUNK - 2, (_NCHUNK - 2) % 3, _NCHUNK % 3, False, 1)
    phase(_NCHUNK - 1, (_NCHUNK - 1) % 3, (_NCHUNK + 1) % 3, False, 2)
    phase(_NCHUNK, _NCHUNK % 3, (_NCHUNK + 2) % 3, False, 3)

    plsc.subcore_barrier()
    pltpu.sync_copy(agg_sh.at[pl.ds(row0, _RPT)],
                    out_hbm.at[cid, pl.ds(row0, _RPT)])


_sc_agg = functools.partial(
    pl.kernel,
    mesh=plsc.VectorSubcoreMesh(core_axis_name="c", subcore_axis_name="s"),
    out_type=jax.ShapeDtypeStruct((_NC, _NPAD, D), jnp.float32),
    scratch_types=(
        [pltpu.VMEM((1, _CHUNK), jnp.int32) for _ in range(2 * _DEPTH)]
        + [pltpu.VMEM((_CHUNK, D), jnp.float32) for _ in range(_DEPTH)]
        + [pltpu.VMEM_SHARED((_NPAD, D), jnp.float32)]
        + [pltpu.SemaphoreType.DMA for _ in range(3 * _DEPTH)]
    ),
)(_sc_agg_body)


def _in_body(x_ref, w_ref, b_ref, h_ref, r_ref):
    h = jnp.dot(x_ref[...], w_ref[...],
                preferred_element_type=jnp.float32) + b_ref[...]
    h_ref[...] = h
    r_ref[...] = jnp.maximum(h, 0.0)


_in_call = pl.pallas_call(
    _in_body,
    out_shape=[jax.ShapeDtypeStruct((N, D), jnp.float32),
               jax.ShapeDtypeStruct((N, D), jnp.float32)],
)


_MLP_BLK = 2000


def _mlp_body(s_ref, h_ref, a_ref, w1_ref, b1_ref, g_ref, be_ref,
              w2_ref, b2_ref, ho_ref, ro_ref):
    h = h_ref[...]
    z = s_ref[0] * h + a_ref[0] + a_ref[1]
    t = jnp.dot(z, w1_ref[...], preferred_element_type=jnp.float32) + b1_ref[...]
    mu = jnp.mean(t, axis=-1, keepdims=True)
    c = t - mu
    var = jnp.mean(c * c, axis=-1, keepdims=True)
    t = c * lax.rsqrt(var + 1e-5) * g_ref[...] + be_ref[...]
    t = jnp.maximum(t, 0.0)
    u = jnp.dot(t, w2_ref[...], preferred_element_type=jnp.float32) + b2_ref[...]
    hn = h + u
    ho_ref[...] = hn
    ro_ref[...] = jnp.maximum(hn, 0.0)


_mlp_call = pl.pallas_call(
    _mlp_body,
    grid=(N // _MLP_BLK,),
    in_specs=[
        pl.BlockSpec(memory_space=pltpu.SMEM),
        pl.BlockSpec((_MLP_BLK, D), lambda i: (i, 0)),
        pl.BlockSpec((_NC, _MLP_BLK, D), lambda i: (0, i, 0)),
        pl.BlockSpec((D, 2 * D), lambda i: (0, 0)),
        pl.BlockSpec((1, 2 * D), lambda i: (0, 0)),
        pl.BlockSpec((1, 2 * D), lambda i: (0, 0)),
        pl.BlockSpec((1, 2 * D), lambda i: (0, 0)),
        pl.BlockSpec((2 * D, D), lambda i: (0, 0)),
        pl.BlockSpec((1, D), lambda i: (0, 0)),
    ],
    out_specs=[
        pl.BlockSpec((_MLP_BLK, D), lambda i: (i, 0)),
        pl.BlockSpec((_MLP_BLK, D), lambda i: (i, 0)),
    ],
    out_shape=[jax.ShapeDtypeStruct((N, D), jnp.float32),
               jax.ShapeDtypeStruct((N, D), jnp.float32)],
)


def _mlp3_body(s_ref, h_ref, a_ref, w1_ref, b1_ref, g_ref, be_ref,
               w2_ref, b2_ref, b2d_ref, wo1_ref, bo1_ref, wo2_ref, bo2_ref,
               o_ref, acc_ref):
    i = pl.program_id(0)
    h = h_ref[...]
    z = s_ref[0] * h + a_ref[0] + a_ref[1]
    t = jnp.dot(z, w1_ref[...], preferred_element_type=jnp.float32) + b1_ref[...]
    mu = jnp.mean(t, axis=-1, keepdims=True)
    c = t - mu
    var = jnp.mean(c * c, axis=-1, keepdims=True)
    t = c * lax.rsqrt(var + 1e-5) * g_ref[...] + be_ref[...]
    t = jnp.maximum(t, 0.0)
    u = jnp.dot(t, w2_ref[...], preferred_element_type=jnp.float32) + b2_ref[...]
    hn = h + u
    seg = b2d_ref[0]
    gid = lax.broadcasted_iota(jnp.int32, (G, _MLP_BLK), 0)
    onehot = jnp.where(gid == seg, 1.0, 0.0)
    pooled = jnp.dot(onehot, hn, preferred_element_type=jnp.float32)

    @pl.when(i == 0)
    def _():
        acc_ref[...] = pooled

    @pl.when(i > 0)
    def _():
        acc_ref[...] = acc_ref[...] + pooled

    @pl.when(i == N // _MLP_BLK - 1)
    def _():
        t2 = jnp.dot(acc_ref[...], wo1_ref[...],
                     preferred_element_type=jnp.float32) + bo1_ref[...]
        t2 = jnp.maximum(t2, 0.0)
        o_ref[...] = jnp.dot(t2, wo2_ref[...],
                             preferred_element_type=jnp.float32) + bo2_ref[...]


_mlp3_call = pl.pallas_call(
    _mlp3_body,
    grid=(N // _MLP_BLK,),
    in_specs=[
        pl.BlockSpec(memory_space=pltpu.SMEM),
        pl.BlockSpec((_MLP_BLK, D), lambda i: (i, 0)),
        pl.BlockSpec((_NC, _MLP_BLK, D), lambda i: (0, i, 0)),
        pl.BlockSpec((D, 2 * D), lambda i: (0, 0)),
        pl.BlockSpec((1, 2 * D), lambda i: (0, 0)),
        pl.BlockSpec((1, 2 * D), lambda i: (0, 0)),
        pl.BlockSpec((1, 2 * D), lambda i: (0, 0)),
        pl.BlockSpec((2 * D, D), lambda i: (0, 0)),
        pl.BlockSpec((1, D), lambda i: (0, 0)),
        pl.BlockSpec((1, 1, _MLP_BLK), lambda i: (i, 0, 0)),
        pl.BlockSpec((D, 2 * D), lambda i: (0, 0)),
        pl.BlockSpec((1, 2 * D), lambda i: (0, 0)),
        pl.BlockSpec((2 * D, D), lambda i: (0, 0)),
        pl.BlockSpec((1, D), lambda i: (0, 0)),
    ],
    out_specs=pl.BlockSpec((G, D), lambda i: (0, 0)),
    out_shape=jax.ShapeDtypeStruct((G, D), jnp.float32),
    scratch_shapes=[pltpu.VMEM((G, D), jnp.float32)],
)


def kernel(x, edge_index, batch, W_in, b_in, eps, W1, b1, gamma, beta,
           W2, b2, Wo1, bo1, Wo2, bo2):
    src = edge_index[0].astype(jnp.int32).reshape(_NW, _NCHUNK, 1, _CHUNK)
    dst = edge_index[1].astype(jnp.int32).reshape(_NW, _NCHUNK, 1, _CHUNK)
    batch2d = batch.astype(jnp.int32).reshape(N // _MLP_BLK, 1, _MLP_BLK)

    h, r = _in_call(x, W_in, b_in.reshape(1, D))
    for i in range(2):
        agg = _sc_agg(r, src, dst)
        scale = (1.0 + eps[i]).reshape(1)
        h, r = _mlp_call(scale, h, agg, W1[i], b1[i].reshape(1, 2 * D),
                         gamma[i].reshape(1, 2 * D), beta[i].reshape(1, 2 * D),
                         W2[i], b2[i].reshape(1, D))
    agg = _sc_agg(r, src, dst)
    scale = (1.0 + eps[2]).reshape(1)
    out = _mlp3_call(scale, h, agg, W1[2], b1[2].reshape(1, 2 * D),
                     gamma[2].reshape(1, 2 * D), beta[2].reshape(1, 2 * D),
                     W2[2], b2[2].reshape(1, D), batch2d,
                     Wo1, bo1.reshape(1, 2 * D), Wo2, bo2.reshape(1, D))
    return out.reshape(-1)
